# Initial kernel scaffold; baseline (speedup 1.0000x reference)
import jax
import jax.numpy as jnp
from jax.experimental import pallas as pl


def kernel(x, t, table, W1, b1, W2, b2):
    raise NotImplementedError("write your pallas kernel here")


# speedup vs baseline: 1.5353x; 1.5353x over previous
import functools
import math

import jax
import jax.numpy as jnp
from jax import lax
from jax.experimental import pallas as pl
from jax.experimental.pallas import tpu as pltpu
from jax.experimental.pallas import tpu_sc as plsc

VOCAB = 100000
DIM = 1024
NUM_HEADS = 16
B = 4
L = 4096

_HALF = DIM // 2
_HH = (DIM // NUM_HEADS) // 2

_NC = 2
_NS = 16
_NW = _NC * _NS
_ROWS = B * L
_RPW = _ROWS // _NW
_CHUNK = 32
_NCHUNK = _RPW // _CHUNK


def _sc_gather_body(idx_hbm, table_hbm, out_hbm, idx_v, buf0, buf1,
                    gsem0, gsem1, psem0, psem1):
    wid = lax.axis_index("s") * _NC + lax.axis_index("c")
    base = wid * _RPW
    pltpu.sync_copy(idx_hbm.at[pl.ds(base, _RPW)], idx_v)

    bufs = (buf0, buf1)
    gsems = (gsem0, gsem1)
    psems = (psem0, psem1)
    gather = [None, None]
    put = [None, None]

    gather[0] = pltpu.async_copy(
        table_hbm.at[idx_v.at[pl.ds(0, _CHUNK)]], bufs[0], gsems[0])
    for c in range(_NCHUNK):
        b = c % 2
        nb = (c + 1) % 2
        if c + 1 < _NCHUNK:
            if put[nb] is not None:
                put[nb].wait()
            gather[nb] = pltpu.async_copy(
                table_hbm.at[idx_v.at[pl.ds((c + 1) * _CHUNK, _CHUNK)]],
                bufs[nb], gsems[nb])
        gather[b].wait()
        put[b] = pltpu.async_copy(
            bufs[b], out_hbm.at[pl.ds(base + c * _CHUNK, _CHUNK)], psems[b])
    put[0].wait()
    put[1].wait()


@functools.cache
def _sc_gather():
    return pl.kernel(
        _sc_gather_body,
        mesh=plsc.VectorSubcoreMesh(core_axis_name="c", subcore_axis_name="s"),
        out_type=jax.ShapeDtypeStruct((_ROWS, DIM), jnp.float32),
        scratch_types=[
            pltpu.VMEM((_RPW,), jnp.int32),
            pltpu.VMEM((_CHUNK, DIM), jnp.float32),
            pltpu.VMEM((_CHUNK, DIM), jnp.float32),
            pltpu.SemaphoreType.DMA,
            pltpu.SemaphoreType.DMA,
            pltpu.SemaphoreType.DMA,
            pltpu.SemaphoreType.DMA,
        ],
    )


def _tc_body(t_ref, w1_ref, b1_ref, w2_ref, b2_ref, c_ref, cos_ref, sin_ref):
    t = t_ref[:]
    i = lax.broadcasted_iota(jnp.int32, (1, _HALF), 1).astype(jnp.float32)
    freqs = jnp.exp(-(math.log(10000.0) / _HALF) * i)
    args = t * freqs
    emb = jnp.concatenate([jnp.sin(args), jnp.cos(args)], axis=-1)
    h1 = jnp.dot(emb, w1_ref[:], preferred_element_type=jnp.float32) + b1_ref[:]
    h1 = h1 * (1.0 / (1.0 + jnp.exp(-h1)))
    c_ref[:] = (jnp.dot(h1, w2_ref[:], preferred_element_type=jnp.float32)
                + b2_ref[:])
    pos = lax.broadcasted_iota(jnp.int32, (L, _HH), 0).astype(jnp.float32)
    j = lax.broadcasted_iota(jnp.int32, (L, _HH), 1).astype(jnp.float32)
    ang = pos * jnp.exp(-(math.log(10000.0) / _HH) * j)
    cos_ref[:] = jnp.cos(ang)
    sin_ref[:] = jnp.sin(ang)


def _tc_small(t2, W1, b1r, W2, b2r):
    return pl.pallas_call(
        _tc_body,
        out_shape=(
            jax.ShapeDtypeStruct((B, DIM), jnp.float32),
            jax.ShapeDtypeStruct((L, _HH), jnp.float32),
            jax.ShapeDtypeStruct((L, _HH), jnp.float32),
        ),
    )(t2, W1, b1r, W2, b2r)


def kernel(x, t, table, W1, b1, W2, b2):
    idx = x.reshape(_ROWS).astype(jnp.int32)
    h = _sc_gather()(idx, table).reshape(B, L, DIM)
    c, cos, sin = _tc_small(t.reshape(B, 1), W1, b1.reshape(1, DIM),
                            W2, b2.reshape(1, DIM))
    return (h, c, cos[None, None], sin[None, None])


# speedup vs baseline: 1.5435x; 1.0053x over previous
import functools
import math

import jax
import jax.numpy as jnp
from jax import lax
from jax.experimental import pallas as pl
from jax.experimental.pallas import tpu as pltpu
from jax.experimental.pallas import tpu_sc as plsc

VOCAB = 100000
DIM = 1024
NUM_HEADS = 16
B = 4
L = 4096

_HALF = DIM // 2
_HH = (DIM // NUM_HEADS) // 2

_NC = 2
_NS = 16
_NW = _NC * _NS
_ROWS = B * L
_RPW = _ROWS // _NW
_CHUNK = 32
_NCHUNK = _RPW // _CHUNK
_NBUF = 3


def _sc_gather_body(idx_hbm, table_hbm, out_hbm, idx_v, *rest):
    bufs = rest[:_NBUF]
    gsems = rest[_NBUF:2 * _NBUF]
    psems = rest[2 * _NBUF:3 * _NBUF]
    wid = lax.axis_index("s") * _NC + lax.axis_index("c")
    base = wid * _RPW
    pltpu.sync_copy(idx_hbm.at[pl.ds(base, _RPW)], idx_v)

    gather = [None] * _NBUF
    put = [None] * _NBUF

    for c in range(_NBUF - 1):
        gather[c] = pltpu.async_copy(
            table_hbm.at[idx_v.at[pl.ds(c * _CHUNK, _CHUNK)]],
            bufs[c], gsems[c])
    for c in range(_NCHUNK):
        b = c % _NBUF
        nxt = c + _NBUF - 1
        if nxt < _NCHUNK:
            nb = nxt % _NBUF
            if put[nb] is not None:
                put[nb].wait()
            gather[nb] = pltpu.async_copy(
                table_hbm.at[idx_v.at[pl.ds(nxt * _CHUNK, _CHUNK)]],
                bufs[nb], gsems[nb])
        gather[b].wait()
        put[b] = pltpu.async_copy(
            bufs[b], out_hbm.at[pl.ds(base + c * _CHUNK, _CHUNK)], psems[b])
    for b in range(_NBUF):
        put[b].wait()


@functools.cache
def _sc_gather():
    return pl.kernel(
        _sc_gather_body,
        mesh=plsc.VectorSubcoreMesh(core_axis_name="c", subcore_axis_name="s"),
        out_type=jax.ShapeDtypeStruct((_ROWS, DIM), jnp.float32),
        scratch_types=(
            [pltpu.VMEM((_RPW,), jnp.int32)]
            + [pltpu.VMEM((_CHUNK, DIM), jnp.float32)] * _NBUF
            + [pltpu.SemaphoreType.DMA] * (2 * _NBUF)
        ),
    )


def _tc_body(t_ref, w1_ref, b1_ref, w2_ref, b2_ref, c_ref, cos_ref, sin_ref):
    t = t_ref[:]
    i = lax.broadcasted_iota(jnp.int32, (1, _HALF), 1).astype(jnp.float32)
    freqs = jnp.exp(-(math.log(10000.0) / _HALF) * i)
    args = t * freqs
    emb = jnp.concatenate([jnp.sin(args), jnp.cos(args)], axis=-1)
    h1 = jnp.dot(emb, w1_ref[:], preferred_element_type=jnp.float32) + b1_ref[:]
    h1 = h1 * (1.0 / (1.0 + jnp.exp(-h1)))
    c_ref[:] = (jnp.dot(h1, w2_ref[:], preferred_element_type=jnp.float32)
                + b2_ref[:])
    pos = lax.broadcasted_iota(jnp.int32, (L, _HH), 0).astype(jnp.float32)
    j = lax.broadcasted_iota(jnp.int32, (L, _HH), 1).astype(jnp.float32)
    ang = pos * jnp.exp(-(math.log(10000.0) / _HH) * j)
    cos_ref[:] = jnp.cos(ang)
    sin_ref[:] = jnp.sin(ang)


def _tc_small(t2, W1, b1r, W2, b2r):
    return pl.pallas_call(
        _tc_body,
        out_shape=(
            jax.ShapeDtypeStruct((B, DIM), jnp.float32),
            jax.ShapeDtypeStruct((L, _HH), jnp.float32),
            jax.ShapeDtypeStruct((L, _HH), jnp.float32),
        ),
    )(t2, W1, b1r, W2, b2r)


def kernel(x, t, table, W1, b1, W2, b2):
    idx = x.reshape(_ROWS).astype(jnp.int32)
    h = _sc_gather()(idx, table).reshape(B, L, DIM)
    c, cos, sin = _tc_small(t.reshape(B, 1), W1, b1.reshape(1, DIM),
                            W2, b2.reshape(1, DIM))
    return (h, c, cos[None, None], sin[None, None])
